# Initial kernel scaffold; baseline (speedup 1.0000x reference)
import jax
import jax.numpy as jnp
from jax.experimental import pallas as pl


def kernel(x, edge_weight, edge_index, W1, W2, W3, W4, Wc1, Wc2, Wcomp, W5, W7):
    raise NotImplementedError("write your pallas kernel here")


# speedup vs baseline: 6.8580x; 6.8580x over previous
import functools

import jax
import jax.numpy as jnp
from jax import lax
from jax.experimental import pallas as pl
from jax.experimental.pallas import tpu as pltpu
from jax.experimental.pallas import tpu_sc as plsc

P = 128
NC = 2
NS = 16
NW = NC * NS
CH = 128


def _relu(v):
    return jnp.maximum(v, 0.0)


def _dot(a, b):
    return lax.dot_general(a, b, (((1,), (0,)), ((), ())),
                           preferred_element_type=jnp.float32,
                           precision=lax.Precision.HIGHEST)


def _chunk_split(E):
    assert E % CH == 0
    NCHUNK = E // CH
    F = NCHUNK // NW
    R = NCHUNK - F * NW
    return F, R


@functools.lru_cache(maxsize=None)
def _build_sc_deg(E, ND):
    F, R = _chunk_split(E)
    ZL = ND // NS
    assert ZL % 128 == 0

    mesh = plsc.VectorSubcoreMesh(core_axis_name="c", subcore_axis_name="s")

    @functools.partial(
        pl.kernel,
        out_type=(
            jax.ShapeDtypeStruct((NC, 1, ND), jnp.float32),
            jax.ShapeDtypeStruct((NC, 1, ND), jnp.float32),
        ),
        mesh=mesh,
        scratch_types=[
            pltpu.VMEM_SHARED((ND,), jnp.float32),
            pltpu.VMEM_SHARED((ND,), jnp.float32),
            pltpu.VMEM((ZL,), jnp.float32),
            pltpu.VMEM((CH,), jnp.float32),
            pltpu.VMEM((2, CH), jnp.int32),
        ],
    )
    def deg_kernel(ew_hbm, src_hbm, dst_hbm, outi_hbm, outo_hbm,
                   acc_i, acc_o, zb, ewb, sib):
        cid = lax.axis_index("c")
        sid = lax.axis_index("s")
        wid = cid * NS + sid
        z16 = jnp.zeros((16,), jnp.float32)

        def zfill(i, carry):
            zb[pl.ds(i * 16, 16)] = z16
            return carry
        lax.fori_loop(0, ZL // 16, zfill, None)
        off0 = pl.multiple_of(sid * ZL, 128)
        pltpu.sync_copy(zb, acc_i.at[pl.ds(off0, ZL)])
        pltpu.sync_copy(zb, acc_o.at[pl.ds(off0, ZL)])
        plsc.subcore_barrier()

        def chunk(j):
            off = pl.multiple_of((wid + j * NW) * CH, 128)
            pltpu.sync_copy(ew_hbm.at[pl.ds(off, CH)], ewb)
            pltpu.sync_copy(dst_hbm.at[pl.ds(off, CH)], sib.at[0])
            pltpu.sync_copy(src_hbm.at[pl.ds(off, CH)], sib.at[1])
            pltpu.sync_copy(ewb, acc_i.at[sib.at[0]], add=True)
            pltpu.sync_copy(ewb, acc_o.at[sib.at[1]], add=True)

        def body(j, carry):
            chunk(j)
            return carry
        lax.fori_loop(0, F, body, None)
        if R:
            @pl.when(wid < R)
            def _extra():
                chunk(F)
        plsc.subcore_barrier()
        pltpu.sync_copy(acc_i.at[pl.ds(off0, ZL)],
                        outi_hbm.at[cid, 0, pl.ds(off0, ZL)])
        pltpu.sync_copy(acc_o.at[pl.ds(off0, ZL)],
                        outo_hbm.at[cid, 0, pl.ds(off0, ZL)])

    return deg_kernel


@functools.lru_cache(maxsize=None)
def _build_sc_agg(E, NP_):
    F, R = _chunk_split(E)
    RT = NP_ // NS
    ZR = 128
    assert NP_ % (NS * ZR) == 0

    mesh = plsc.VectorSubcoreMesh(core_axis_name="c", subcore_axis_name="s")

    @functools.partial(
        pl.kernel,
        out_type=jax.ShapeDtypeStruct((NC, 2, NP_, P), jnp.float32),
        mesh=mesh,
        scratch_types=[
            pltpu.VMEM_SHARED((NP_, P), jnp.float32),
            pltpu.VMEM((ZR, P), jnp.float32),
            pltpu.VMEM((CH,), jnp.int32),
            pltpu.VMEM((1, CH), jnp.int32),
            pltpu.VMEM((CH, P), jnp.float32),
            pltpu.SemaphoreType.DMA,
        ],
    )
    def agg_kernel(mu_hbm, src_hbm, dst_hbm, out_hbm, acc, zb, gib, sib, rows,
                   sem):
        cid = lax.axis_index("c")
        sid = lax.axis_index("s")
        wid = cid * NS + sid
        z16 = jnp.zeros((16,), jnp.float32)

        def zfill(i, carry):
            for j in range(P // 16):
                zb[i, pl.ds(j * 16, 16)] = z16
            return carry
        lax.fori_loop(0, ZR, zfill, None)

        row0 = pl.multiple_of(sid * RT, 128)
        for d in range(2):
            g_hbm = src_hbm if d == 0 else dst_hbm
            s_hbm = dst_hbm if d == 0 else src_hbm
            for k in range(RT // ZR):
                pltpu.sync_copy(zb, acc.at[pl.ds(row0 + k * ZR, ZR)])
            plsc.subcore_barrier()

            def chunk(j):
                off = pl.multiple_of((wid + j * NW) * CH, 128)
                pltpu.sync_copy(g_hbm.at[pl.ds(off, CH)], gib)
                pltpu.sync_copy(s_hbm.at[pl.ds(off, CH)], sib.at[0])
                pltpu.async_copy(mu_hbm.at[gib], rows, sem).wait()
                pltpu.sync_copy(rows, acc.at[sib.at[0]], add=True)

            def body(j, carry):
                chunk(j)
                return carry
            lax.fori_loop(0, F, body, None)
            if R:
                @pl.when(wid < R)
                def _extra():
                    chunk(F)
            plsc.subcore_barrier()
            pltpu.sync_copy(acc.at[pl.ds(row0, RT)],
                            out_hbm.at[cid, d, pl.ds(row0, RT)])
            plsc.subcore_barrier()

    return agg_kernel


def _fold_call(W1r, W4r, W2T, W3T, Wc1aT, Wc1bT, Wc1cT):
    T_ = W1r.shape[0]

    def body(W1r_ref, W4r_ref, W2T_ref, W3T_ref, a_ref, b_ref, c_ref,
             u1_ref, u3_ref, M2T_ref):
        for t in range(T_):
            u1_ref[t] = _dot(W1r_ref[t], a_ref[t])
            r4 = _relu(W4r_ref[t])
            u3_ref[t] = _dot(_dot(r4, W3T_ref[t]), c_ref[t])
            M2T_ref[t] = _dot(W2T_ref[t], b_ref[t])

    return pl.pallas_call(
        body,
        out_shape=(
            jax.ShapeDtypeStruct((T_, 1, P), jnp.float32),
            jax.ShapeDtypeStruct((T_, 1, P), jnp.float32),
            jax.ShapeDtypeStruct((T_, P, P), jnp.float32),
        ),
    )(W1r, W4r, W2T, W3T, Wc1aT, Wc1bT, Wc1cT)


def _wspec(shape):
    nd = len(shape)
    return pl.BlockSpec(shape, lambda i, _nd=nd: (0,) * _nd)


def _tc0_call(x, dip, dop, u1, u3, Wc2T, WcAT, WcBT, B):
    N = x.shape[0]

    def body(x_ref, dip_ref, dop_ref, u1_ref, u3_ref, Wc2T_ref, WcAT_ref,
             WcBT_ref, mu_ref):
        xb = x_ref[...]
        u1v = u1_ref[...]
        u3v = u3_ref[...]
        hi = xb * u1v + (dip_ref[0] + dip_ref[1]) * u3v
        mi = _relu(_dot(_relu(hi), Wc2T_ref[...]))
        ho = xb * u1v + (dop_ref[0] + dop_ref[1]) * u3v
        mo = _relu(_dot(_relu(ho), Wc2T_ref[...]))
        mu_ref[...] = _relu(_dot(mi, WcAT_ref[...]) + _dot(mo, WcBT_ref[...]))

    return pl.pallas_call(
        body,
        grid=(N // B,),
        in_specs=[
            pl.BlockSpec((B, 1), lambda i: (i, 0)),
            pl.BlockSpec((NC, B, 1), lambda i: (0, i, 0)),
            pl.BlockSpec((NC, B, 1), lambda i: (0, i, 0)),
            _wspec((1, P)), _wspec((1, P)), _wspec((P, P)),
            _wspec((P, P)), _wspec((P, P)),
        ],
        out_specs=pl.BlockSpec((B, P), lambda i: (i, 0)),
        out_shape=jax.ShapeDtypeStruct((N, P), jnp.float32),
        compiler_params=pltpu.CompilerParams(
            dimension_semantics=("arbitrary",)),
    )(x, dip, dop, u1, u3, Wc2T, WcAT, WcBT)


def _tc_step_call(x, dip, dop, aggp, u1, u3, M2T, Wc2T, WcAT, WcBT, B):
    N = x.shape[0]

    def body(x_ref, dip_ref, dop_ref, aggp_ref, u1_ref, u3_ref, M2T_ref,
             Wc2T_ref, WcAT_ref, WcBT_ref, mu_ref, pool_ref):
        xb = x_ref[...]
        u1v = u1_ref[...]
        u3v = u3_ref[...]
        m2 = M2T_ref[...]
        hi = xb * u1v + (dip_ref[0] + dip_ref[1]) * u3v \
            + _dot(aggp_ref[0, 0] + aggp_ref[1, 0], m2)
        mi = _relu(_dot(_relu(hi), Wc2T_ref[...]))
        ho = xb * u1v + (dop_ref[0] + dop_ref[1]) * u3v \
            + _dot(aggp_ref[0, 1] + aggp_ref[1, 1], m2)
        mo = _relu(_dot(_relu(ho), Wc2T_ref[...]))
        mu_blk = _relu(_dot(mi, WcAT_ref[...]) + _dot(mo, WcBT_ref[...]))
        mu_ref[...] = mu_blk

        @pl.when(pl.program_id(0) == 0)
        def _init():
            pool_ref[...] = jnp.zeros((1, P), jnp.float32)
        pool_ref[...] += jnp.sum(mu_blk, axis=0, keepdims=True)

    return pl.pallas_call(
        body,
        grid=(N // B,),
        in_specs=[
            pl.BlockSpec((B, 1), lambda i: (i, 0)),
            pl.BlockSpec((NC, B, 1), lambda i: (0, i, 0)),
            pl.BlockSpec((NC, B, 1), lambda i: (0, i, 0)),
            pl.BlockSpec((NC, 2, B, P), lambda i: (0, 0, i, 0)),
            _wspec((1, P)), _wspec((1, P)), _wspec((P, P)), _wspec((P, P)),
            _wspec((P, P)), _wspec((P, P)),
        ],
        out_specs=(
            pl.BlockSpec((B, P), lambda i: (i, 0)),
            pl.BlockSpec((1, P), lambda i: (0, 0)),
        ),
        out_shape=(
            jax.ShapeDtypeStruct((N, P), jnp.float32),
            jax.ShapeDtypeStruct((1, P), jnp.float32),
        ),
        compiler_params=pltpu.CompilerParams(
            dimension_semantics=("arbitrary",)),
    )(x, dip, dop, aggp, u1, u3, M2T, Wc2T, WcAT, WcBT)


def _tc_out_call(mu, pool, W7T, w5aT, w5bT, B):
    N = mu.shape[0]

    def body(mu_ref, pool_ref, W7T_ref, w5aT_ref, w5bT_ref, out_ref):
        c = _dot(_relu(pool_ref[...]), w5aT_ref[...])
        tt = _relu(_dot(mu_ref[...], W7T_ref[...]))
        out_ref[...] = _relu(c + _dot(tt, w5bT_ref[...]))

    return pl.pallas_call(
        body,
        grid=(N // B,),
        in_specs=[
            pl.BlockSpec((B, P), lambda i: (i, 0)),
            _wspec((1, P)), _wspec((P, P)), _wspec((P, 1)), _wspec((P, 1)),
        ],
        out_specs=pl.BlockSpec((B, 1), lambda i: (i, 0)),
        out_shape=jax.ShapeDtypeStruct((N, 1), jnp.float32),
        compiler_params=pltpu.CompilerParams(
            dimension_semantics=("arbitrary",)),
    )(mu, pool, W7T, w5aT, w5bT)


def kernel(x, edge_weight, edge_index, W1, W2, W3, W4, Wc1, Wc2, Wcomp, W5, W7):
    N = x.shape[0]
    E = edge_index.shape[1]
    T_ = W1.shape[0]

    ei = edge_index.astype(jnp.int32)
    src = ei[0]
    dst = ei[1]
    ew = edge_weight.reshape(E).astype(jnp.float32)

    W1r = jnp.transpose(W1, (0, 2, 1))
    W4r = jnp.transpose(W4, (0, 2, 1))
    W2T = jnp.transpose(W2, (0, 2, 1))
    W3T = jnp.transpose(W3, (0, 2, 1))
    Wc1T = jnp.transpose(Wc1, (0, 2, 1))
    u1, u3, M2T = _fold_call(W1r, W4r, W2T, W3T,
                             Wc1T[:, 0 * P:1 * P, :],
                             Wc1T[:, 1 * P:2 * P, :],
                             Wc1T[:, 2 * P:3 * P, :])
    Wc2T = jnp.transpose(Wc2, (0, 2, 1))
    WcAT = Wcomp[:, :P].T
    WcBT = Wcomp[:, P:].T
    W7T = W7.T
    w5aT = W5[:, :P].T
    w5bT = W5[:, P:].T

    ND = ((N + NS * 128 - 1) // (NS * 128)) * (NS * 128)
    degi, dego = _build_sc_deg(E, ND)(ew, src, dst)
    dip = degi[:, 0, :N].reshape(NC, N, 1)
    dop = dego[:, 0, :N].reshape(NC, N, 1)

    B = 1000 if N % 1000 == 0 else N

    mu = _tc0_call(x, dip, dop, u1[0], u3[0], Wc2T[0], WcAT, WcBT, B)
    pool = None
    for t in range(1, T_):
        aggp = _build_sc_agg(E, ND)(mu, src, dst)
        mu, pool = _tc_step_call(x, dip, dop, aggp, u1[t], u3[t], M2T[t],
                                 Wc2T[t], WcAT, WcBT, B)
    out = _tc_out_call(mu, pool, W7T, w5aT, w5bT, B)
    return out.reshape(N)
